# Initial kernel scaffold; baseline (speedup 1.0000x reference)
import jax
import jax.numpy as jnp
from jax.experimental import pallas as pl


def kernel(inlayer, edge_index, edge_values, w0):
    raise NotImplementedError("write your pallas kernel here")


# speedup vs baseline: 5.5951x; 5.5951x over previous
import functools

import jax
import jax.numpy as jnp
from jax import lax
from jax.experimental import pallas as pl
from jax.experimental.pallas import tpu as pltpu
from jax.experimental.pallas import tpu_sc as plsc

N_NODES = 10000
N_EDGES = 320000
D_FEAT = 128

NC = 2
NS = 16
NW = NC * NS
E_PER_W = N_EDGES // NW
CHUNK = 128
N_FULL = E_PER_W // CHUNK
REM = E_PER_W - N_FULL * CHUNK
ROWS_PER_TEC = 624
ROWS_TAIL = N_NODES - NS * ROWS_PER_TEC
FV = D_FEAT // 16


def _sc_body(x_hbm, rows_hbm, cols_hbm, ev_hbm, out_hbm,
             acc_sh, gbuf, colv, rowv, evv, g16, col16, row16, ev16, sem):
    cid = lax.axis_index("c")
    sid = lax.axis_index("s")
    wid = sid * NC + cid

    def zrow(r, _):
        for f in range(FV):
            gbuf[r, pl.ds(f * 16, 16)] = jnp.zeros((16,), jnp.float32)
        return _
    lax.fori_loop(0, CHUNK, zrow, None)

    base = sid * ROWS_PER_TEC
    for j in range(4):
        pltpu.sync_copy(gbuf, acc_sh.at[pl.ds(base + j * CHUNK, CHUNK)])
    pltpu.sync_copy(gbuf.at[pl.ds(0, ROWS_PER_TEC - 4 * CHUNK)],
                    acc_sh.at[pl.ds(base + 4 * CHUNK, ROWS_PER_TEC - 4 * CHUNK)])

    @pl.when(sid == 0)
    def _zero_tail():
        pltpu.sync_copy(gbuf.at[pl.ds(0, ROWS_TAIL)],
                        acc_sh.at[pl.ds(NS * ROWS_PER_TEC, ROWS_TAIL)])
    plsc.subcore_barrier()

    ebase = wid * E_PER_W

    def chunk(ci, _):
        eb = ebase + ci * CHUNK
        pltpu.sync_copy(cols_hbm.at[pl.ds(eb, CHUNK)], colv)
        pltpu.sync_copy(rows_hbm.at[pl.ds(eb, CHUNK)], rowv)
        pltpu.sync_copy(ev_hbm.at[pl.ds(eb, CHUNK)], evv)
        pltpu.async_copy(x_hbm.at[colv], gbuf, sem).wait()

        def scale(g, _):
            evg = evv[pl.ds(g * 16, 16)]
            for i in range(16):
                e = g * 16 + i
                s = evg[i]
                for f in range(FV):
                    gbuf[e, pl.ds(f * 16, 16)] = gbuf[e, pl.ds(f * 16, 16)] * s
            return _
        lax.fori_loop(0, CHUNK // 16, scale, None)

        pltpu.sync_copy(gbuf, acc_sh.at[rowv], add=True)
        return _
    lax.fori_loop(0, N_FULL, chunk, None)

    eb = ebase + N_FULL * CHUNK
    pltpu.sync_copy(cols_hbm.at[pl.ds(eb, REM)], col16)
    pltpu.sync_copy(rows_hbm.at[pl.ds(eb, REM)], row16)
    pltpu.sync_copy(ev_hbm.at[pl.ds(eb, REM)], ev16)
    pltpu.async_copy(x_hbm.at[col16], g16, sem).wait()

    evg16 = ev16[pl.ds(0, 16)]
    for i in range(REM):
        s = evg16[i]
        for f in range(FV):
            g16[i, pl.ds(f * 16, 16)] = g16[i, pl.ds(f * 16, 16)] * s
    pltpu.sync_copy(g16, acc_sh.at[row16], add=True)

    plsc.subcore_barrier()
    pltpu.sync_copy(acc_sh.at[pl.ds(base, ROWS_PER_TEC)],
                    out_hbm.at[cid, pl.ds(base, ROWS_PER_TEC)])

    @pl.when(sid == 0)
    def _out_tail():
        pltpu.sync_copy(acc_sh.at[pl.ds(NS * ROWS_PER_TEC, ROWS_TAIL)],
                        out_hbm.at[cid, pl.ds(NS * ROWS_PER_TEC, ROWS_TAIL)])


@jax.jit
def _sc_spmm(x, rows, cols, ev):
    mesh = plsc.VectorSubcoreMesh(core_axis_name="c", subcore_axis_name="s",
                                  num_cores=NC, num_subcores=NS)
    return pl.kernel(
        _sc_body,
        out_type=jax.ShapeDtypeStruct((NC, N_NODES, D_FEAT), jnp.float32),
        mesh=mesh,
        scratch_types=[
            pltpu.VMEM_SHARED((N_NODES, D_FEAT), jnp.float32),
            pltpu.VMEM((CHUNK, D_FEAT), jnp.float32),
            pltpu.VMEM((CHUNK,), jnp.int32),
            pltpu.VMEM((CHUNK,), jnp.int32),
            pltpu.VMEM((CHUNK,), jnp.float32),
            pltpu.VMEM((REM, D_FEAT), jnp.float32),
            pltpu.VMEM((REM,), jnp.int32),
            pltpu.VMEM((REM,), jnp.int32),
            pltpu.VMEM((REM,), jnp.float32),
            pltpu.SemaphoreType.DMA,
        ],
    )(x, rows, cols, ev)


def _combine_body(p_ref, w_ref, o_ref):
    o_ref[...] = jnp.tanh((p_ref[0] + p_ref[1]) * w_ref[...])


@jax.jit
def _tc_combine(partials, w0):
    blk = 2000
    return pl.pallas_call(
        _combine_body,
        grid=(N_NODES // blk,),
        in_specs=[
            pl.BlockSpec((NC, blk, D_FEAT), lambda i: (0, i, 0)),
            pl.BlockSpec((1, D_FEAT), lambda i: (0, 0)),
        ],
        out_specs=pl.BlockSpec((blk, D_FEAT), lambda i: (i, 0)),
        out_shape=jax.ShapeDtypeStruct((N_NODES, D_FEAT), jnp.float32),
    )(partials, w0)


def kernel(inlayer, edge_index, edge_values, w0):
    rows = edge_index[0].astype(jnp.int32)
    cols = edge_index[1].astype(jnp.int32)
    partials = _sc_spmm(inlayer, rows, cols, edge_values)
    return _tc_combine(partials, w0)
